# baseline (speedup 1.0000x reference)
import functools

import jax
import jax.numpy as jnp
from jax import lax
from jax.experimental import pallas as pl
from jax.experimental.pallas import tpu as pltpu
from jax.experimental.pallas import tpu_sc as plsc

N = 50000
D = 512
E = 50000
RATIO = 0.5
K = max(int(N * RATIO), 1)

NS = 16
PERM_PAD = 25088
TPERM = PERM_PAD // NS
NODE_PAD = 50176
TNODE = NODE_PAD // NS
NCHUNK = TNODE // 16
GROWS = 32
GCHUNKS = TPERM // GROWS
PARK = K


def _matvec_block(x_ref, w_ref, o_ref):
    o_ref[...] = jnp.dot(x_ref[...], w_ref[...],
                         preferred_element_type=jnp.float32)


def _scores_matvec(x, W):
    BLK = 2000
    out = pl.pallas_call(
        _matvec_block,
        grid=(N // BLK,),
        in_specs=[
            pl.BlockSpec((BLK, D), lambda i: (i, 0)),
            pl.BlockSpec((D, 1), lambda i: (0, 0)),
        ],
        out_specs=pl.BlockSpec((BLK, 1), lambda i: (i, 0)),
        out_shape=jax.ShapeDtypeStruct((N, 1), jnp.float32),
    )(x, W)
    return out[:, 0]


def _iota16():
    return lax.broadcasted_iota(jnp.int32, (16,), 0)


def _pool_body(x_hbm, perm_g_hbm, perm_s_hbm, rank_s_hbm, ei_hbm, ej_hbm,
               lexc_hbm, offs_hbm, cnt_hbm,
               xpool_hbm, eiout_hbm, ejout_hbm, batchout_hbm,
               enc_vm, ei_vm, ej_vm, bufi_vm, bufj_vm, zbuf_vm,
               pidx_vm, rankv_vm, lex_vm, off_vm, cnt_vm, pg_vm, gbuf_vm,
               drain_vm, enc_sp,
               sem_g0, sem_g1, sem_o0, sem_o1, sem_a, sem_b):
    cid = lax.axis_index("c")
    sid = lax.axis_index("s")
    iota = _iota16()

    @pl.when(cid == 0)
    def _core0():
        def zb(m, carry):
            zbuf_vm[pl.ds(16 * m, 16)] = jnp.zeros((16,), jnp.int32)
            return carry
        lax.fori_loop(0, NCHUNK, zb, 0)
        pltpu.sync_copy(zbuf_vm.at[pl.ds(0, TNODE)],
                        enc_sp.at[pl.ds(sid * TNODE, TNODE)])
        pltpu.sync_copy(zbuf_vm.at[pl.ds(0, TPERM)],
                        batchout_hbm.at[pl.ds(sid * TPERM, TPERM)])
        plsc.subcore_barrier()

        pltpu.sync_copy(rank_s_hbm.at[sid], rankv_vm)
        for j in range(14):
            pltpu.sync_copy(perm_s_hbm.at[sid, j], pidx_vm)
            pltpu.sync_copy(rankv_vm.at[j], enc_sp.at[pidx_vm])
        plsc.subcore_barrier()

        pltpu.sync_copy(enc_sp, enc_vm)
        pltpu.sync_copy(ei_hbm.at[sid], ei_vm)
        pltpu.sync_copy(ej_hbm.at[sid], ej_vm)

        nbase = sid * TNODE
        stripe = NCHUNK * iota
        pltpu.sync_copy(lexc_hbm.at[sid], lex_vm)
        pltpu.sync_copy(offs_hbm.at[sid], off_vm)
        pltpu.sync_copy(cnt_hbm.at[sid], cnt_vm)
        lane_excl = lex_vm[...]
        offs = off_vm[...]
        count = cnt_vm[...]

        def comp_body(m, run):
            idx = nbase + stripe + m
            encc = plsc.load_gather(enc_vm, [idx])
            keep = (encc == 0) & (idx < N)
            ki = keep.astype(jnp.int32)
            pos = lane_excl + run
            lidx = stripe + m
            eic = plsc.load_gather(ei_vm, [lidx])
            ejc = plsc.load_gather(ej_vm, [lidx])
            ri = jnp.maximum(plsc.load_gather(enc_vm, [eic]) - 1, 0)
            rj = jnp.maximum(plsc.load_gather(enc_vm, [ejc]) - 1, 0)
            plsc.store_scatter(bufi_vm, [pos], ri, mask=keep)
            plsc.store_scatter(bufj_vm, [pos], rj, mask=keep)
            return run + ki
        lax.fori_loop(0, NCHUNK, comp_body, jnp.zeros((16,), jnp.int32))

        def sc_body(u, carry):
            lpos = 16 * u + iota
            cond = lpos < count
            gp = jnp.where(cond, offs + lpos, PARK + (lpos & 63))
            pltpu.async_copy(bufi_vm.at[pl.ds(16 * u, 16)],
                             eiout_hbm.at[gp], sem_a)
            pltpu.async_copy(bufj_vm.at[pl.ds(16 * u, 16)],
                             ejout_hbm.at[gp], sem_b)

            @pl.when(u >= 8)
            def _():
                pltpu.make_async_copy(eiout_hbm.at[pl.ds(0, 16)],
                                      drain_vm, sem_a).wait()
                pltpu.make_async_copy(ejout_hbm.at[pl.ds(0, 16)],
                                      drain_vm, sem_b).wait()
            return carry
        lax.fori_loop(0, NCHUNK, sc_body, 0)
        for _ in range(8):
            pltpu.make_async_copy(eiout_hbm.at[pl.ds(0, 16)],
                                  drain_vm, sem_a).wait()
            pltpu.make_async_copy(ejout_hbm.at[pl.ds(0, 16)],
                                  drain_vm, sem_b).wait()

    @pl.when(cid == 1)
    def _core1():
        pltpu.sync_copy(perm_g_hbm.at[sid], pg_vm)
        base = sid * TPERM
        gsems = [sem_g0, sem_g1]
        osems = [sem_o0, sem_o1]
        pltpu.async_copy(x_hbm.at[pg_vm.at[pl.ds(0, GROWS)]],
                         gbuf_vm.at[0], sem_g0)
        for c in range(GCHUNKS):
            s = c & 1
            pltpu.make_async_copy(x_hbm.at[pg_vm.at[pl.ds(0, GROWS)]],
                                  gbuf_vm.at[s], gsems[s]).wait()
            if c + 1 < GCHUNKS:
                if c >= 1:
                    pltpu.make_async_copy(
                        gbuf_vm.at[1 - s],
                        xpool_hbm.at[pl.ds(0, GROWS)], osems[1 - s]).wait()
                pltpu.async_copy(
                    x_hbm.at[pg_vm.at[pl.ds((c + 1) * GROWS, GROWS)]],
                    gbuf_vm.at[1 - s], gsems[1 - s])
            pltpu.async_copy(gbuf_vm.at[s],
                             xpool_hbm.at[pl.ds(base + c * GROWS, GROWS)],
                             osems[s])
        pltpu.make_async_copy(gbuf_vm.at[0],
                              xpool_hbm.at[pl.ds(0, GROWS)], sem_o0).wait()
        pltpu.make_async_copy(gbuf_vm.at[1],
                              xpool_hbm.at[pl.ds(0, GROWS)], sem_o1).wait()


@functools.partial(jax.jit, static_argnums=())
def _pool_sc(x, perm, ei, ej):
    pad88 = jnp.arange(88, dtype=jnp.int32)
    perm_g = jnp.concatenate([perm, jnp.zeros((88,), jnp.int32)])
    perm_g = perm_g.reshape(NS, TPERM)
    perm_s = jnp.concatenate([perm, N + pad88]).reshape(NS, 14, 112)
    rank_s = (jnp.arange(PERM_PAD, dtype=jnp.int32) + 1).reshape(
        NS, 14, 112)
    zpad = jnp.zeros((NODE_PAD - E,), jnp.int32)
    ei_p = jnp.concatenate([ei, zpad]).reshape(NS, TNODE)
    ej_p = jnp.concatenate([ej, zpad]).reshape(NS, TNODE)

    strip = jnp.minimum(jnp.arange(256, dtype=jnp.int32) * 196 + 196, N)
    strip = strip - jnp.minimum(jnp.arange(256, dtype=jnp.int32) * 196, N)
    hist = jnp.zeros((256,), jnp.int32).at[perm // 196].add(1)
    keepc = strip - hist
    csum = jnp.cumsum(keepc) - keepc
    tile_tot = jnp.sum(keepc.reshape(NS, 16), axis=1)
    tile_off = (jnp.cumsum(tile_tot) - tile_tot).astype(jnp.int32)
    lexc = (csum.reshape(NS, 16)
            - tile_off[:, None]).astype(jnp.int32)
    offs_in = jnp.broadcast_to(tile_off[:, None], (NS, 16)).astype(jnp.int32)
    cnt_in = jnp.broadcast_to(tile_tot.astype(jnp.int32)[:, None], (NS, 16))

    mesh = plsc.VectorSubcoreMesh(core_axis_name="c", subcore_axis_name="s")
    fn = pl.kernel(
        _pool_body,
        mesh=mesh,
        compiler_params=pltpu.CompilerParams(needs_layout_passes=False),
        out_type=[
            jax.ShapeDtypeStruct((PERM_PAD, D), jnp.float32),
            jax.ShapeDtypeStruct((PERM_PAD,), jnp.int32),
            jax.ShapeDtypeStruct((PERM_PAD,), jnp.int32),
            jax.ShapeDtypeStruct((PERM_PAD,), jnp.int32),
        ],
        scratch_types=[
            pltpu.VMEM((NODE_PAD,), jnp.int32),
            pltpu.VMEM((TNODE,), jnp.int32),
            pltpu.VMEM((TNODE,), jnp.int32),
            pltpu.VMEM((TNODE,), jnp.int32),
            pltpu.VMEM((TNODE,), jnp.int32),
            pltpu.VMEM((TNODE,), jnp.int32),
            pltpu.VMEM((112,), jnp.int32),
            pltpu.VMEM((14, 112), jnp.int32),
            pltpu.VMEM((16,), jnp.int32),
            pltpu.VMEM((16,), jnp.int32),
            pltpu.VMEM((16,), jnp.int32),
            pltpu.VMEM((TPERM,), jnp.int32),
            pltpu.VMEM((2, GROWS, D), jnp.float32),
            pltpu.VMEM((16,), jnp.int32),
            pltpu.VMEM_SHARED((NODE_PAD,), jnp.int32),
            pltpu.SemaphoreType.DMA,
            pltpu.SemaphoreType.DMA,
            pltpu.SemaphoreType.DMA,
            pltpu.SemaphoreType.DMA,
            pltpu.SemaphoreType.DMA,
            pltpu.SemaphoreType.DMA,
        ],
    )
    xpool, eio, ejo, bat = fn(x, perm_g, perm_s, rank_s, ei_p, ej_p,
                              lexc, offs_in, cnt_in)
    return xpool[:K], eio[:E - K], ejo[:E - K], bat[:K]


def kernel(x, edge_index, batch, W):
    scores = _scores_matvec(x, W)
    edge_index_i = edge_index[0]
    edge_index_j = edge_index[1]
    neighbor_scores = scores[edge_index_j]
    aggregated = jnp.zeros_like(scores).at[edge_index_i].add(neighbor_scores)
    scores = scores + aggregated
    vals, perm = jax.lax.top_k(scores, K)
    x_pool, new_ei, new_ej, batch_pool = _pool_sc(
        x, perm, edge_index_i, edge_index_j)
    edge_index_pool = jnp.stack([new_ei, new_ej], axis=0)
    return (x_pool, edge_index_pool, perm, batch_pool, vals)
